# Initial kernel scaffold; baseline (speedup 1.0000x reference)
import jax
import jax.numpy as jnp
from jax.experimental import pallas as pl


def kernel(x, a, i, W1, b1, g1, be1, m1, v1, W2, b2, g2, be2, m2, v2, Wd1, bd1, Wd2, bd2, Wd3, bd3):
    raise NotImplementedError("write your pallas kernel here")


# speedup vs baseline: 4.8451x; 4.8451x over previous
import functools

import jax
import jax.numpy as jnp
from jax import lax
from jax.experimental import pallas as pl
from jax.experimental.pallas import tpu as pltpu
from jax.experimental.pallas import tpu_sc as plsc

N = 10000
E = 320000
D = 128
H = 64
G = 100
NPG = 100
EPS = 1e-3

NUM_CORES = 2
NUM_SUBCORES = 16
NW = NUM_CORES * NUM_SUBCORES
CH = 128
EPW = 10112
NCHUNK = EPW // CH
EPAD = NW * EPW
NP = 10112
ZR = NP // NUM_SUBCORES


def _edge_agg_body(h_hbm, src_hbm, dst_hbm, zero_hbm, out_hbm,
                   src_v, dst_v, rows_v, wb_v, acc_sh, sem):
    c = lax.axis_index("c")
    s = lax.axis_index("s")
    wid = s * NUM_CORES + c

    pltpu.sync_copy(zero_hbm, wb_v)
    pltpu.sync_copy(wb_v, acc_sh.at[pl.ds(s * ZR, ZR)])
    plsc.subcore_barrier()

    base0 = wid * EPW

    def body(j, carry):
        b = pl.multiple_of(base0 + j * CH, 8)
        pltpu.sync_copy(src_hbm.at[pl.ds(b, CH)], src_v)
        pltpu.sync_copy(dst_hbm.at[pl.ds(b, CH)], dst_v)
        pltpu.async_copy(h_hbm.at[src_v], rows_v, sem).wait()
        pltpu.sync_copy(rows_v, acc_sh.at[dst_v], add=True)
        return carry

    lax.fori_loop(0, NCHUNK, body, 0)
    plsc.subcore_barrier()

    wsl = pl.ds(s * ZR, ZR)
    pltpu.sync_copy(acc_sh.at[wsl], wb_v)
    pltpu.sync_copy(wb_v, out_hbm.at[c, wsl])


@functools.cache
def _build_edge_agg():
    return pl.kernel(
        _edge_agg_body,
        mesh=plsc.VectorSubcoreMesh(core_axis_name="c", subcore_axis_name="s"),
        out_type=jax.ShapeDtypeStruct((NUM_CORES, NP, H), jnp.float32),
        scratch_types=[
            pltpu.VMEM((CH,), jnp.int32),
            pltpu.VMEM((CH,), jnp.int32),
            pltpu.VMEM((CH, H), jnp.float32),
            pltpu.VMEM((ZR, H), jnp.float32),
            pltpu.VMEM_SHARED((NP, H), jnp.float32),
            pltpu.SemaphoreType.DMA,
        ],
        compiler_params=pltpu.CompilerParams(use_tc_tiling_on_sc=False),
    )


def _edge_agg(h, srcp, dstp, zeros):
    return _build_edge_agg()(h, srcp, dstp, zeros)


def _elu(x):
    return jnp.where(x > 0, x, jnp.exp(jnp.minimum(x, 0.0)) - 1.0)


def _dense1_body(x_ref, w_ref, c_ref, o_ref):
    o_ref[...] = jnp.dot(x_ref[...], w_ref[...],
                         preferred_element_type=jnp.float32) + c_ref[...]


def _dense2_body(p_ref, w_ref, c_ref, o_ref):
    t = _elu(p_ref[0] + p_ref[1])
    o_ref[...] = jnp.dot(t, w_ref[...],
                         preferred_element_type=jnp.float32) + c_ref[...]


def _combine_body(p_ref, o_ref):
    o_ref[...] = _elu(p_ref[0] + p_ref[1])


def _head_body(g_ref, w1_ref, b1_ref, w2_ref, b2_ref, w3_ref, b3_ref, o_ref):
    t = jax.nn.relu(jnp.dot(g_ref[...], w1_ref[...],
                            preferred_element_type=jnp.float32) + b1_ref[...])
    t = jax.nn.relu(jnp.dot(t, w2_ref[...],
                            preferred_element_type=jnp.float32) + b2_ref[...])
    o_ref[...] = jax.nn.sigmoid(jnp.dot(t, w3_ref[...],
                                        preferred_element_type=jnp.float32)
                                + b3_ref[...])


def _dense1(x, w, c):
    return pl.pallas_call(
        _dense1_body,
        out_shape=jax.ShapeDtypeStruct((NP, H), jnp.float32),
    )(x, w, c)


def _dense2(p, w, c):
    return pl.pallas_call(
        _dense2_body,
        out_shape=jax.ShapeDtypeStruct((NP, H), jnp.float32),
    )(p, w, c)


def _combine(p):
    return pl.pallas_call(
        _combine_body,
        out_shape=jax.ShapeDtypeStruct((NP, H), jnp.float32),
    )(p)


def _head(g, w1, b1, w2, b2, w3, b3):
    return pl.pallas_call(
        _head_body,
        out_shape=jax.ShapeDtypeStruct((G, 1), jnp.float32),
    )(g, w1, b1, w2, b2, w3, b3)


def kernel(x, a, i, W1, b1, g1, be1, m1, v1, W2, b2, g2, be2, m2, v2,
           Wd1, bd1, Wd2, bd2, Wd3, bd3):
    inv1 = g1 / jnp.sqrt(v1 + EPS)
    W1f = W1 * inv1[None, :]
    c1 = ((b1 - m1) * inv1 + be1)[None, :]
    inv2 = g2 / jnp.sqrt(v2 + EPS)
    W2f = W2 * inv2[None, :]
    c2 = ((b2 - m2) * inv2 + be2)[None, :]

    pad = EPAD - E
    srcp = jnp.concatenate([a[0], jnp.zeros((pad,), jnp.int32)])
    dstp = jnp.concatenate([a[1], jnp.full((pad,), N, jnp.int32)])
    zeros = jnp.zeros((ZR, H), jnp.float32)

    xp = jnp.pad(x, ((0, NP - N), (0, 0)))
    h1 = _dense1(xp, W1f, c1)
    p1 = _edge_agg(h1, srcp, dstp, zeros)
    h2 = _dense2(p1, W2f, c2)
    p2 = _edge_agg(h2, srcp, dstp, zeros)
    e2 = _combine(p2)
    g = e2[:N].reshape(G, NPG * H)
    return _head(g, Wd1, bd1[None, :], Wd2, bd2[None, :], Wd3, bd3[None, :])


# speedup vs baseline: 5.2795x; 1.0897x over previous
import functools

import jax
import jax.numpy as jnp
from jax import lax
from jax.experimental import pallas as pl
from jax.experimental.pallas import tpu as pltpu
from jax.experimental.pallas import tpu_sc as plsc

N = 10000
E = 320000
D = 128
H = 64
G = 100
NPG = 100
EPS = 1e-3

NUM_CORES = 2
NUM_SUBCORES = 16
NW = NUM_CORES * NUM_SUBCORES
CH = 128
NCHUNK = 80
EPW = NCHUNK * CH
EPAD = NW * EPW
NP = 10112
ZR = NP // NUM_SUBCORES


def _edge_agg_body(h_hbm, src_hbm, dst_hbm, zero_hbm, out_hbm,
                   src2_v, dst2_v, rows0_v, rows1_v, wb_v, acc_sh,
                   sem0, sem1):
    c = lax.axis_index("c")
    s = lax.axis_index("s")
    wid = s * NUM_CORES + c

    rsl = pl.ds(wid * NCHUNK, NCHUNK)
    pltpu.sync_copy(src_hbm.at[rsl], src2_v)
    pltpu.sync_copy(dst_hbm.at[rsl], dst2_v)

    pltpu.sync_copy(zero_hbm, wb_v)
    pltpu.sync_copy(wb_v, acc_sh.at[pl.ds(s * ZR, ZR)])
    plsc.subcore_barrier()

    pltpu.async_copy(h_hbm.at[src2_v.at[0]], rows0_v, sem0)

    def body(k, carry):
        j = 2 * k
        pltpu.async_copy(h_hbm.at[src2_v.at[j + 1]], rows1_v, sem1)
        pltpu.make_async_copy(h_hbm.at[src2_v.at[j]], rows0_v, sem0).wait()
        pltpu.sync_copy(rows0_v, acc_sh.at[dst2_v.at[j]], add=True)

        @pl.when(k < NCHUNK // 2 - 1)
        def _():
            pltpu.async_copy(h_hbm.at[src2_v.at[j + 2]], rows0_v, sem0)

        pltpu.make_async_copy(h_hbm.at[src2_v.at[j + 1]], rows1_v, sem1).wait()
        pltpu.sync_copy(rows1_v, acc_sh.at[dst2_v.at[j + 1]], add=True)
        return carry

    lax.fori_loop(0, NCHUNK // 2, body, 0)
    plsc.subcore_barrier()

    wsl = pl.ds(s * ZR, ZR)
    pltpu.sync_copy(acc_sh.at[wsl], wb_v)
    pltpu.sync_copy(wb_v, out_hbm.at[c, wsl])


@functools.cache
def _build_edge_agg():
    return pl.kernel(
        _edge_agg_body,
        mesh=plsc.VectorSubcoreMesh(core_axis_name="c", subcore_axis_name="s"),
        out_type=jax.ShapeDtypeStruct((NUM_CORES, NP, H), jnp.float32),
        scratch_types=[
            pltpu.VMEM((NCHUNK, CH), jnp.int32),
            pltpu.VMEM((NCHUNK, CH), jnp.int32),
            pltpu.VMEM((CH, H), jnp.float32),
            pltpu.VMEM((CH, H), jnp.float32),
            pltpu.VMEM((ZR, H), jnp.float32),
            pltpu.VMEM_SHARED((NP, H), jnp.float32),
            pltpu.SemaphoreType.DMA,
            pltpu.SemaphoreType.DMA,
        ],
        compiler_params=pltpu.CompilerParams(use_tc_tiling_on_sc=False),
    )


def _edge_agg(h, srcp, dstp, zeros):
    return _build_edge_agg()(h, srcp, dstp, zeros)


def _elu(x):
    return jnp.where(x > 0, x, jnp.exp(jnp.minimum(x, 0.0)) - 1.0)


def _dense1_body(x_ref, w_ref, c_ref, o_ref):
    o_ref[...] = jnp.dot(x_ref[...], w_ref[...],
                         preferred_element_type=jnp.float32) + c_ref[...]


def _dense2_body(p_ref, w_ref, c_ref, o_ref):
    t = _elu(p_ref[0] + p_ref[1])
    o_ref[...] = jnp.dot(t, w_ref[...],
                         preferred_element_type=jnp.float32) + c_ref[...]


def _combine_body(p_ref, o_ref):
    o_ref[...] = _elu(p_ref[0] + p_ref[1])


def _head_body(g_ref, w1_ref, b1_ref, w2_ref, b2_ref, w3_ref, b3_ref, o_ref):
    t = jax.nn.relu(jnp.dot(g_ref[...], w1_ref[...],
                            preferred_element_type=jnp.float32) + b1_ref[...])
    t = jax.nn.relu(jnp.dot(t, w2_ref[...],
                            preferred_element_type=jnp.float32) + b2_ref[...])
    o_ref[...] = jax.nn.sigmoid(jnp.dot(t, w3_ref[...],
                                        preferred_element_type=jnp.float32)
                                + b3_ref[...])


def _dense1(x, w, c):
    return pl.pallas_call(
        _dense1_body,
        out_shape=jax.ShapeDtypeStruct((NP, H), jnp.float32),
    )(x, w, c)


def _dense2(p, w, c):
    return pl.pallas_call(
        _dense2_body,
        out_shape=jax.ShapeDtypeStruct((NP, H), jnp.float32),
    )(p, w, c)


def _combine(p):
    return pl.pallas_call(
        _combine_body,
        out_shape=jax.ShapeDtypeStruct((NP, H), jnp.float32),
    )(p)


def _head(g, w1, b1, w2, b2, w3, b3):
    return pl.pallas_call(
        _head_body,
        out_shape=jax.ShapeDtypeStruct((G, 1), jnp.float32),
    )(g, w1, b1, w2, b2, w3, b3)


def kernel(x, a, i, W1, b1, g1, be1, m1, v1, W2, b2, g2, be2, m2, v2,
           Wd1, bd1, Wd2, bd2, Wd3, bd3):
    inv1 = g1 / jnp.sqrt(v1 + EPS)
    W1f = W1 * inv1[None, :]
    c1 = ((b1 - m1) * inv1 + be1)[None, :]
    inv2 = g2 / jnp.sqrt(v2 + EPS)
    W2f = W2 * inv2[None, :]
    c2 = ((b2 - m2) * inv2 + be2)[None, :]

    pad = EPAD - E
    srcp = jnp.concatenate([a[0], jnp.zeros((pad,), jnp.int32)])
    srcp = srcp.reshape(EPAD // CH, CH)
    dstp = jnp.concatenate([a[1], jnp.full((pad,), N, jnp.int32)])
    dstp = dstp.reshape(EPAD // CH, CH)
    zeros = jnp.zeros((ZR, H), jnp.float32)

    xp = jnp.pad(x, ((0, NP - N), (0, 0)))
    h1 = _dense1(xp, W1f, c1)
    p1 = _edge_agg(h1, srcp, dstp, zeros)
    h2 = _dense2(p1, W2f, c2)
    p2 = _edge_agg(h2, srcp, dstp, zeros)
    e2 = _combine(p2)
    g = e2[:N].reshape(G, NPG * H)
    return _head(g, Wd1, bd1[None, :], Wd2, bd2[None, :], Wd3, bd3[None, :])
